# baseline (speedup 1.0000x reference)
import functools

import jax
import jax.numpy as jnp
from jax.experimental import pallas as pl
from jax.experimental.pallas import tpu as pltpu

DEPTH = 2
DIM = 512
HEADS = 8
DH = DIM // HEADS
B = 8
N = 28 * 28
WINDOW = 28 * 7
CBS = 28 * 14
STRIDE = 28 * 7
SBS = 28 * 14
MLP = DIM * 4

f32 = jnp.float32
bf16 = jnp.bfloat16

DN = (((1,), (1,)), ((), ()))


def _layer(tok, consts, g, wq, wk, wv, wg, wkc, wvc, wo, lng, lnb, w1, b1,
           w2, b2):
    band16, inv_colhalf16, diff16, ones_col, rowhead = consts
    x = tok
    xn = x * jax.lax.rsqrt(jnp.mean(x * x, axis=-1, keepdims=True) + 1e-6)
    xnb = (xn * g).astype(bf16)

    q32 = jnp.dot(xnb, wq, preferred_element_type=f32)
    k32 = jnp.dot(xnb, wk, preferred_element_type=f32)
    v32 = jnp.dot(xnb, wv, preferred_element_type=f32)
    gates = jax.nn.sigmoid(jnp.dot(xnb, wg, preferred_element_type=f32))
    scale = DH ** -0.5
    qsb = (q32 * scale).astype(bf16)
    kb = k32.astype(bf16)
    vb = v32.astype(bf16)

    pool_k = jnp.concatenate([
        jnp.mean(k32[0:CBS], axis=0, keepdims=True),
        jnp.mean(k32[STRIDE:STRIDE + CBS], axis=0, keepdims=True),
        jnp.mean(k32[2 * STRIDE:2 * STRIDE + CBS], axis=0, keepdims=True),
    ], axis=0)
    pool_v = jnp.concatenate([
        jnp.mean(v32[0:CBS], axis=0, keepdims=True),
        jnp.mean(v32[STRIDE:STRIDE + CBS], axis=0, keepdims=True),
        jnp.mean(v32[2 * STRIDE:2 * STRIDE + CBS], axis=0, keepdims=True),
    ], axis=0)
    sel_k = jnp.concatenate([
        jnp.mean(k32[:SBS], axis=0, keepdims=True),
        jnp.mean(k32[SBS:], axis=0, keepdims=True),
    ], axis=0)
    poolk16 = pool_k.astype(bf16)
    poolv16 = pool_v.astype(bf16)

    sel_kt = sel_k.T
    sk_bd = jnp.concatenate(
        [sel_kt * jnp.where(rowhead == h, 1.0, 0.0) for h in range(HEADS)],
        axis=1)
    imp_all = jnp.dot(q32, sk_bd, preferred_element_type=f32)

    outs = []
    for h in range(HEADS):
        sl = slice(h * DH, (h + 1) * DH)
        sim = jax.lax.dot_general(qsb[:, sl], kb[:, sl], DN,
                                  preferred_element_type=f32).astype(bf16)
        e = jnp.exp(sim)

        sel1f = jnp.where(imp_all[:, 2 * h + 1:2 * h + 2] >
                          imp_all[:, 2 * h:2 * h + 1], 1.0, 0.0).astype(bf16)
        msel16 = inv_colhalf16 + sel1f * diff16

        ew = e * band16
        es = e * msel16
        vext = jnp.concatenate([vb[:, sl], ones_col], axis=1)
        o_w = jnp.dot(ew, vext, preferred_element_type=f32)
        o_s = jnp.dot(es, vext, preferred_element_type=f32)
        gw_col = gates[:, 3 * h + 2:3 * h + 3] / o_w[:, DH:DH + 1]
        gs_col = gates[:, 3 * h + 1:3 * h + 2] / o_s[:, DH:DH + 1]
        out_h = gw_col * o_w[:, :DH] + gs_col * o_s[:, :DH]

        kc = jnp.dot(poolk16[:, sl], wkc,
                     preferred_element_type=f32).astype(bf16)
        vc = jnp.dot(poolv16[:, sl], wvc,
                     preferred_element_type=f32).astype(bf16)
        sim_c = jax.lax.dot_general(qsb[:, sl], kc, DN,
                                    preferred_element_type=f32)
        p_c = jnp.exp(sim_c)
        p_c = (p_c / jnp.sum(p_c, axis=-1, keepdims=True)).astype(bf16)
        out_c = jnp.dot(p_c, vc, preferred_element_type=f32)

        outs.append(gates[:, 3 * h:3 * h + 1] * out_c + out_h)

    attn = jnp.concatenate(outs, axis=-1).astype(bf16)

    y = jnp.dot(attn, wo, preferred_element_type=f32) + x
    mu = jnp.mean(y, axis=-1, keepdims=True)
    var = jnp.mean(jnp.square(y - mu), axis=-1, keepdims=True)
    ln = (y - mu) * jax.lax.rsqrt(var + 1e-5) * lng + lnb
    hmid = jax.lax.dot_general(ln.astype(bf16), w1, DN,
                               preferred_element_type=f32)
    hmid = jax.nn.gelu((hmid + b1).astype(bf16))
    return jax.lax.dot_general(hmid, w2, DN,
                               preferred_element_type=f32) + b2 + y


def _net_body(x_ref, pos_ref, g_ref, wq_ref, wk_ref, wv_ref, wg_ref, wkc_ref,
              wvc_ref, wo_ref, lng_ref, lnb_ref, w1_ref, b1_ref, w2_ref,
              b2_ref, out_ref):
    rows = jax.lax.broadcasted_iota(jnp.int32, (N, N), 0)
    cols = jax.lax.broadcasted_iota(jnp.int32, (N, N), 1)
    band16 = jnp.where(jnp.abs(rows - cols) < WINDOW, 1.0, 0.0).astype(bf16)
    inv_colhalf16 = jnp.where(cols < SBS, 1.0, 0.0).astype(bf16)
    diff16 = jnp.where(cols >= SBS, 1.0, -1.0).astype(bf16)
    ones_col = jnp.ones((N, 1), bf16)
    rowhead = jax.lax.broadcasted_iota(jnp.int32, (DIM, 1), 0) // DH
    consts = (band16, inv_colhalf16, diff16, ones_col, rowhead)

    tok = jnp.transpose(x_ref[0] + pos_ref[...])
    for i in range(DEPTH):
        tok = _layer(tok, consts, g_ref[i], wq_ref[i], wk_ref[i], wv_ref[i],
                     wg_ref[i], wkc_ref[i], wvc_ref[i], wo_ref[i], lng_ref[i],
                     lnb_ref[i], w1_ref[i], b1_ref[i], w2_ref[i], b2_ref[i])
    out_ref[0] = jnp.transpose(tok)


@functools.partial(jax.jit, static_argnames=())
def kernel(x, pos_emb, g, Wq, Wk, Wv, Wkc, Wvc, Wg, Wo, ln_g, ln_b,
           W1, b1, W2, b2):
    b, c, h, w = x.shape
    xr = x.reshape(b, c, N)

    whole = lambda *dims: pl.BlockSpec(dims, lambda bi: (0,) * len(dims))
    out = pl.pallas_call(
        _net_body,
        grid=(B,),
        in_specs=[
            pl.BlockSpec((1, DIM, N), lambda bi: (bi, 0, 0)),
            whole(1, N),
            whole(DEPTH, 1, DIM),
            whole(DEPTH, DIM, DIM), whole(DEPTH, DIM, DIM),
            whole(DEPTH, DIM, DIM), whole(DEPTH, DIM, HEADS * 3),
            whole(DEPTH, DH, DH), whole(DEPTH, DH, DH),
            whole(DEPTH, DIM, DIM),
            whole(DEPTH, 1, DIM), whole(DEPTH, 1, DIM),
            whole(DEPTH, MLP, DIM), whole(DEPTH, 1, MLP),
            whole(DEPTH, DIM, MLP), whole(DEPTH, 1, DIM),
        ],
        out_specs=pl.BlockSpec((1, DIM, N), lambda bi: (bi, 0, 0)),
        out_shape=jax.ShapeDtypeStruct((B, DIM, N), f32),
        compiler_params=pltpu.CompilerParams(
            dimension_semantics=("parallel",)),
    )(xr, pos_emb[:N].reshape(1, N), g.reshape(DEPTH, 1, DIM),
      Wq.astype(bf16), Wk.astype(bf16), Wv.astype(bf16), Wg.astype(bf16),
      Wkc.astype(bf16), Wvc.astype(bf16), Wo.astype(bf16),
      ln_g.reshape(DEPTH, 1, DIM), ln_b.reshape(DEPTH, 1, DIM),
      W1.astype(bf16), b1.reshape(DEPTH, 1, MLP),
      W2.astype(bf16), b2.reshape(DEPTH, 1, DIM))

    return out.reshape(b, c, h, w)


# speedup vs baseline: 1.1696x; 1.1696x over previous
import functools

import jax
import jax.numpy as jnp
from jax.experimental import pallas as pl
from jax.experimental.pallas import tpu as pltpu

DEPTH = 2
DIM = 512
HEADS = 8
DH = DIM // HEADS
B = 8
N = 28 * 28
WINDOW = 28 * 7
CBS = 28 * 14
STRIDE = 28 * 7
SBS = 28 * 14
MLP = DIM * 4

f32 = jnp.float32
bf16 = jnp.bfloat16

DN = (((1,), (1,)), ((), ()))


def _layer_body(tok_ref, g_ref, wq_ref, wk_ref, wv_ref, wg_ref, wkc_ref,
                wvc_ref, wo_ref, lng_ref, lnb_ref, w1t_ref, b1_ref, w2t_ref,
                b2_ref, out_ref):
    x = tok_ref[0]
    xn = x * jax.lax.rsqrt(jnp.mean(x * x, axis=-1, keepdims=True) + 1e-6)
    xnb = (xn * g_ref[...]).astype(bf16)

    q32 = jnp.dot(xnb, wq_ref[...], preferred_element_type=f32)
    k32 = jnp.dot(xnb, wk_ref[...], preferred_element_type=f32)
    v32 = jnp.dot(xnb, wv_ref[...], preferred_element_type=f32)
    gates = jax.nn.sigmoid(jnp.dot(xnb, wg_ref[...],
                                   preferred_element_type=f32))
    scale = DH ** -0.5
    qsb = (q32 * scale).astype(bf16)
    kb = k32.astype(bf16)
    vb = v32.astype(bf16)

    rows = jax.lax.broadcasted_iota(jnp.int32, (N, N), 0)
    cols = jax.lax.broadcasted_iota(jnp.int32, (N, N), 1)
    band16 = jnp.where(jnp.abs(rows - cols) < WINDOW, 1.0, 0.0).astype(bf16)
    inv_colhalf16 = jnp.where(cols < SBS, 1.0, 0.0).astype(bf16)
    diff16 = jnp.where(cols >= SBS, 1.0, -1.0).astype(bf16)
    ones_col = jnp.ones((N, 1), bf16)

    pool_k = jnp.concatenate([
        jnp.mean(k32[0:CBS], axis=0, keepdims=True),
        jnp.mean(k32[STRIDE:STRIDE + CBS], axis=0, keepdims=True),
        jnp.mean(k32[2 * STRIDE:2 * STRIDE + CBS], axis=0, keepdims=True),
    ], axis=0)
    pool_v = jnp.concatenate([
        jnp.mean(v32[0:CBS], axis=0, keepdims=True),
        jnp.mean(v32[STRIDE:STRIDE + CBS], axis=0, keepdims=True),
        jnp.mean(v32[2 * STRIDE:2 * STRIDE + CBS], axis=0, keepdims=True),
    ], axis=0)
    sel_k = jnp.concatenate([
        jnp.mean(k32[:SBS], axis=0, keepdims=True),
        jnp.mean(k32[SBS:], axis=0, keepdims=True),
    ], axis=0)
    poolk16 = pool_k.astype(bf16)
    poolv16 = pool_v.astype(bf16)

    rowhead = jax.lax.broadcasted_iota(jnp.int32, (DIM, 1), 0) // DH
    sel_kt = sel_k.T
    sk_bd = jnp.concatenate(
        [sel_kt * jnp.where(rowhead == h, 1.0, 0.0) for h in range(HEADS)],
        axis=1)
    imp_all = jnp.dot(q32, sk_bd, preferred_element_type=f32)

    outs = []
    for h in range(HEADS):
        sl = slice(h * DH, (h + 1) * DH)
        sim = jax.lax.dot_general(qsb[:, sl], kb[:, sl], DN,
                                  preferred_element_type=f32).astype(bf16)
        e = jnp.exp(sim)

        sel1f = jnp.where(imp_all[:, 2 * h + 1:2 * h + 2] >
                          imp_all[:, 2 * h:2 * h + 1], 1.0, 0.0).astype(bf16)
        msel16 = inv_colhalf16 + sel1f * diff16

        ew = e * band16
        es = e * msel16
        vext = jnp.concatenate([vb[:, sl], ones_col], axis=1)
        o_w = jnp.dot(ew, vext, preferred_element_type=f32)
        o_s = jnp.dot(es, vext, preferred_element_type=f32)
        gw_col = gates[:, 3 * h + 2:3 * h + 3] / o_w[:, DH:DH + 1]
        gs_col = gates[:, 3 * h + 1:3 * h + 2] / o_s[:, DH:DH + 1]
        out_h = gw_col * o_w[:, :DH] + gs_col * o_s[:, :DH]

        kc = jnp.dot(poolk16[:, sl], wkc_ref[...],
                     preferred_element_type=f32).astype(bf16)
        vc = jnp.dot(poolv16[:, sl], wvc_ref[...],
                     preferred_element_type=f32).astype(bf16)
        sim_c = jax.lax.dot_general(qsb[:, sl], kc, DN,
                                    preferred_element_type=f32)
        p_c = jnp.exp(sim_c)
        p_c = (p_c / jnp.sum(p_c, axis=-1, keepdims=True)).astype(bf16)
        out_c = jnp.dot(p_c, vc, preferred_element_type=f32)

        outs.append(gates[:, 3 * h:3 * h + 1] * out_c + out_h)

    attn = jnp.concatenate(outs, axis=-1).astype(bf16)

    y = jnp.dot(attn, wo_ref[...], preferred_element_type=f32) + x
    mu = jnp.mean(y, axis=-1, keepdims=True)
    var = jnp.mean(jnp.square(y - mu), axis=-1, keepdims=True)
    ln = (y - mu) * jax.lax.rsqrt(var + 1e-5) * lng_ref[...] + lnb_ref[...]
    hmid = jnp.dot(ln.astype(bf16), w1t_ref[...], preferred_element_type=f32)
    hmid = jax.nn.gelu((hmid + b1_ref[...]).astype(bf16))
    out_ref[0] = jnp.dot(hmid, w2t_ref[...],
                         preferred_element_type=f32) + b2_ref[...] + y


@functools.partial(jax.jit, static_argnames=())
def kernel(x, pos_emb, g, Wq, Wk, Wv, Wkc, Wvc, Wg, Wo, ln_g, ln_b,
           W1, b1, W2, b2):
    b, c, h, w = x.shape
    x = x + pos_emb[: h * w].reshape(1, 1, h, w)
    tok = x.reshape(b, c, N).transpose(0, 2, 1)

    whole = lambda *dims: pl.BlockSpec(dims, lambda bi: (0,) * len(dims))
    for i in range(DEPTH):
        tok = pl.pallas_call(
            _layer_body,
            grid=(B,),
            in_specs=[
                pl.BlockSpec((1, N, DIM), lambda bi: (bi, 0, 0)),
                whole(1, DIM),
                whole(DIM, DIM), whole(DIM, DIM), whole(DIM, DIM),
                whole(DIM, HEADS * 3),
                whole(DH, DH), whole(DH, DH),
                whole(DIM, DIM),
                whole(1, DIM), whole(1, DIM),
                whole(DIM, MLP), whole(1, MLP),
                whole(MLP, DIM), whole(1, DIM),
            ],
            out_specs=pl.BlockSpec((1, N, DIM), lambda bi: (bi, 0, 0)),
            out_shape=jax.ShapeDtypeStruct((B, N, DIM), f32),
            compiler_params=pltpu.CompilerParams(
                dimension_semantics=("parallel",)),
        )(tok, g[i].reshape(1, DIM),
          Wq[i].astype(bf16), Wk[i].astype(bf16), Wv[i].astype(bf16),
          Wg[i].astype(bf16), Wkc[i].astype(bf16), Wvc[i].astype(bf16),
          Wo[i].astype(bf16),
          ln_g[i].reshape(1, DIM), ln_b[i].reshape(1, DIM),
          W1[i].T.astype(bf16), b1[i].reshape(1, MLP),
          W2[i].T.astype(bf16), b2[i].reshape(1, DIM))

    return tok.transpose(0, 2, 1).reshape(b, c, h, w)


# speedup vs baseline: 1.1729x; 1.0028x over previous
import functools

import jax
import jax.numpy as jnp
from jax.experimental import pallas as pl
from jax.experimental.pallas import tpu as pltpu

DEPTH = 2
DIM = 512
HEADS = 8
DH = DIM // HEADS
B = 8
N = 28 * 28
WINDOW = 28 * 7
CBS = 28 * 14
STRIDE = 28 * 7
SBS = 28 * 14
MLP = DIM * 4

f32 = jnp.float32
bf16 = jnp.bfloat16

DN = (((1,), (1,)), ((), ()))


def _layer_body(first, last, tok_ref, *refs):
    if first:
        pos_ref = refs[0]
        refs = refs[1:]
    (g_ref, wq_ref, wk_ref, wv_ref, wg_ref, wkc_ref, wvc_ref, wo_ref,
     lng_ref, lnb_ref, w1t_ref, b1_ref, w2t_ref, b2_ref, out_ref) = refs
    if first:
        x = jnp.transpose(tok_ref[0] + pos_ref[...])
    else:
        x = tok_ref[0]
    xn = x * jax.lax.rsqrt(jnp.mean(x * x, axis=-1, keepdims=True) + 1e-6)
    xnb = (xn * g_ref[...]).astype(bf16)

    q32 = jnp.dot(xnb, wq_ref[...], preferred_element_type=f32)
    k32 = jnp.dot(xnb, wk_ref[...], preferred_element_type=f32)
    v32 = jnp.dot(xnb, wv_ref[...], preferred_element_type=f32)
    gates = jax.nn.sigmoid(jnp.dot(xnb, wg_ref[...],
                                   preferred_element_type=f32))
    scale = DH ** -0.5
    qsb = (q32 * scale).astype(bf16)
    kb = k32.astype(bf16)
    vb = v32.astype(bf16)

    rows = jax.lax.broadcasted_iota(jnp.int32, (N, N), 0)
    cols = jax.lax.broadcasted_iota(jnp.int32, (N, N), 1)
    band16 = jnp.where(jnp.abs(rows - cols) < WINDOW, 1.0, 0.0).astype(bf16)
    inv_colhalf16 = jnp.where(cols < SBS, 1.0, 0.0).astype(bf16)
    diff16 = jnp.where(cols >= SBS, 1.0, -1.0).astype(bf16)
    ones_col = jnp.ones((N, 1), bf16)

    pool_k = jnp.concatenate([
        jnp.mean(k32[0:CBS], axis=0, keepdims=True),
        jnp.mean(k32[STRIDE:STRIDE + CBS], axis=0, keepdims=True),
        jnp.mean(k32[2 * STRIDE:2 * STRIDE + CBS], axis=0, keepdims=True),
    ], axis=0)
    pool_v = jnp.concatenate([
        jnp.mean(v32[0:CBS], axis=0, keepdims=True),
        jnp.mean(v32[STRIDE:STRIDE + CBS], axis=0, keepdims=True),
        jnp.mean(v32[2 * STRIDE:2 * STRIDE + CBS], axis=0, keepdims=True),
    ], axis=0)
    sel_k = jnp.concatenate([
        jnp.mean(k32[:SBS], axis=0, keepdims=True),
        jnp.mean(k32[SBS:], axis=0, keepdims=True),
    ], axis=0)
    poolk16 = pool_k.astype(bf16)
    poolv16 = pool_v.astype(bf16)

    rowhead = jax.lax.broadcasted_iota(jnp.int32, (DIM, 1), 0) // DH
    sel_kt = sel_k.T
    sk_bd = jnp.concatenate(
        [sel_kt * jnp.where(rowhead == h, 1.0, 0.0) for h in range(HEADS)],
        axis=1)
    imp_all = jnp.dot(q32, sk_bd, preferred_element_type=f32)

    outs = []
    for h in range(HEADS):
        sl = slice(h * DH, (h + 1) * DH)
        sim = jax.lax.dot_general(qsb[:, sl], kb[:, sl], DN,
                                  preferred_element_type=f32).astype(bf16)
        e = jnp.exp(sim)

        sel1f = jnp.where(imp_all[:, 2 * h + 1:2 * h + 2] >
                          imp_all[:, 2 * h:2 * h + 1], 1.0, 0.0).astype(bf16)
        msel16 = inv_colhalf16 + sel1f * diff16

        ew = e * band16
        es = e * msel16
        vext = jnp.concatenate([vb[:, sl], ones_col], axis=1)
        o_w = jnp.dot(ew, vext, preferred_element_type=f32)
        o_s = jnp.dot(es, vext, preferred_element_type=f32)
        gw_col = gates[:, 3 * h + 2:3 * h + 3] / o_w[:, DH:DH + 1]
        gs_col = gates[:, 3 * h + 1:3 * h + 2] / o_s[:, DH:DH + 1]
        out_h = gw_col * o_w[:, :DH] + gs_col * o_s[:, :DH]

        kc = jnp.dot(poolk16[:, sl], wkc_ref[...],
                     preferred_element_type=f32).astype(bf16)
        vc = jnp.dot(poolv16[:, sl], wvc_ref[...],
                     preferred_element_type=f32).astype(bf16)
        sim_c = jax.lax.dot_general(qsb[:, sl], kc, DN,
                                    preferred_element_type=f32)
        p_c = jnp.exp(sim_c)
        p_c = (p_c / jnp.sum(p_c, axis=-1, keepdims=True)).astype(bf16)
        out_c = jnp.dot(p_c, vc, preferred_element_type=f32)

        outs.append(gates[:, 3 * h:3 * h + 1] * out_c + out_h)

    attn = jnp.concatenate(outs, axis=-1).astype(bf16)

    y = jnp.dot(attn, wo_ref[...], preferred_element_type=f32) + x
    mu = jnp.mean(y, axis=-1, keepdims=True)
    var = jnp.mean(jnp.square(y - mu), axis=-1, keepdims=True)
    ln = (y - mu) * jax.lax.rsqrt(var + 1e-5) * lng_ref[...] + lnb_ref[...]
    hmid = jax.lax.dot_general(ln.astype(bf16), w1t_ref[...], DN,
                               preferred_element_type=f32)
    hmid = jax.nn.gelu((hmid + b1_ref[...]).astype(bf16))
    res = jax.lax.dot_general(hmid, w2t_ref[...], DN,
                              preferred_element_type=f32) + b2_ref[...] + y
    if last:
        out_ref[0] = jnp.transpose(res)
    else:
        out_ref[0] = res


@functools.partial(jax.jit, static_argnames=())
def kernel(x, pos_emb, g, Wq, Wk, Wv, Wkc, Wvc, Wg, Wo, ln_g, ln_b,
           W1, b1, W2, b2):
    b, c, h, w = x.shape
    tok = x.reshape(b, c, N)

    whole = lambda *dims: pl.BlockSpec(dims, lambda bi: (0,) * len(dims))
    for i in range(DEPTH):
        first = i == 0
        last = i == DEPTH - 1
        tok_spec = (pl.BlockSpec((1, DIM, N), lambda bi: (bi, 0, 0)) if first
                    else pl.BlockSpec((1, N, DIM), lambda bi: (bi, 0, 0)))
        out_spec = (pl.BlockSpec((1, DIM, N), lambda bi: (bi, 0, 0)) if last
                    else pl.BlockSpec((1, N, DIM), lambda bi: (bi, 0, 0)))
        out_shape = (jax.ShapeDtypeStruct((B, DIM, N), f32) if last
                     else jax.ShapeDtypeStruct((B, N, DIM), f32))
        pos_in = ([pos_emb[:N].reshape(1, N)], [whole(1, N)]) if first \
            else ([], [])
        tok = pl.pallas_call(
            functools.partial(_layer_body, first, last),
            grid=(B,),
            in_specs=[tok_spec] + pos_in[1] + [
                whole(1, DIM),
                whole(DIM, DIM), whole(DIM, DIM), whole(DIM, DIM),
                whole(DIM, HEADS * 3),
                whole(DH, DH), whole(DH, DH),
                whole(DIM, DIM),
                whole(1, DIM), whole(1, DIM),
                whole(MLP, DIM), whole(1, MLP),
                whole(DIM, MLP), whole(1, DIM),
            ],
            out_specs=out_spec,
            out_shape=out_shape,
            compiler_params=pltpu.CompilerParams(
                dimension_semantics=("parallel",)),
        )(tok, *pos_in[0], g[i].reshape(1, DIM),
          Wq[i].astype(bf16), Wk[i].astype(bf16), Wv[i].astype(bf16),
          Wg[i].astype(bf16), Wkc[i].astype(bf16), Wvc[i].astype(bf16),
          Wo[i].astype(bf16),
          ln_g[i].reshape(1, DIM), ln_b[i].reshape(1, DIM),
          W1[i].astype(bf16), b1[i].reshape(1, MLP),
          W2[i].astype(bf16), b2[i].reshape(1, DIM))

    return tok.reshape(b, c, h, w)


# speedup vs baseline: 1.1877x; 1.0126x over previous
import functools

import jax
import jax.numpy as jnp
from jax.experimental import pallas as pl
from jax.experimental.pallas import tpu as pltpu

DEPTH = 2
DIM = 512
HEADS = 8
DH = DIM // HEADS
B = 8
N = 28 * 28
WINDOW = 28 * 7
CBS = 28 * 14
STRIDE = 28 * 7
SBS = 28 * 14
MLP = DIM * 4

f32 = jnp.float32
bf16 = jnp.bfloat16

DN = (((1,), (1,)), ((), ()))


def _layer_body(first, last, tok_ref, *refs):
    if first:
        pos_ref = refs[0]
        refs = refs[1:]
    (wq_ref, wk_ref, wv_ref, wg_ref, wkc_ref, wvc_ref, wo_ref,
     w1t_ref, b1_ref, w2t_ref, b2_ref, out_ref) = refs
    if first:
        x = jnp.transpose(tok_ref[0] + pos_ref[...])
    else:
        x = tok_ref[0]
    xnb = (x * jax.lax.rsqrt(
        jnp.mean(x * x, axis=-1, keepdims=True) + 1e-6)).astype(bf16)

    q32 = jnp.dot(xnb, wq_ref[...], preferred_element_type=f32)
    k32 = jnp.dot(xnb, wk_ref[...], preferred_element_type=f32)
    v32 = jnp.dot(xnb, wv_ref[...], preferred_element_type=f32)
    gates = jax.nn.sigmoid(jnp.dot(xnb, wg_ref[...],
                                   preferred_element_type=f32))
    scale = DH ** -0.5
    qsb = (q32 * scale).astype(bf16)
    kb = k32.astype(bf16)
    vb = v32.astype(bf16)

    rows = jax.lax.broadcasted_iota(jnp.int32, (N, N), 0)
    cols = jax.lax.broadcasted_iota(jnp.int32, (N, N), 1)
    band16 = jnp.where(jnp.abs(rows - cols) < WINDOW, 1.0, 0.0).astype(bf16)
    inv_colhalf16 = jnp.where(cols < SBS, 1.0, 0.0).astype(bf16)
    diff16 = jnp.where(cols >= SBS, 1.0, -1.0).astype(bf16)
    ones_col = jnp.ones((N, 1), bf16)

    pool_k = jnp.concatenate([
        jnp.mean(k32[0:CBS], axis=0, keepdims=True),
        jnp.mean(k32[STRIDE:STRIDE + CBS], axis=0, keepdims=True),
        jnp.mean(k32[2 * STRIDE:2 * STRIDE + CBS], axis=0, keepdims=True),
    ], axis=0)
    pool_v = jnp.concatenate([
        jnp.mean(v32[0:CBS], axis=0, keepdims=True),
        jnp.mean(v32[STRIDE:STRIDE + CBS], axis=0, keepdims=True),
        jnp.mean(v32[2 * STRIDE:2 * STRIDE + CBS], axis=0, keepdims=True),
    ], axis=0)
    sel_k = jnp.concatenate([
        jnp.mean(k32[:SBS], axis=0, keepdims=True),
        jnp.mean(k32[SBS:], axis=0, keepdims=True),
    ], axis=0)
    poolk16 = pool_k.astype(bf16)
    poolv16 = pool_v.astype(bf16)

    rowhead = jax.lax.broadcasted_iota(jnp.int32, (DIM, 1), 0) // DH
    sel_kt = sel_k.T
    sk_bd = jnp.concatenate(
        [sel_kt * jnp.where(rowhead == h, 1.0, 0.0) for h in range(HEADS)],
        axis=1)
    imp_all = jnp.dot(q32, sk_bd, preferred_element_type=f32)

    outs = []
    for h in range(HEADS):
        sl = slice(h * DH, (h + 1) * DH)
        sim = jax.lax.dot_general(qsb[:, sl], kb[:, sl], DN,
                                  preferred_element_type=f32).astype(bf16)
        e = jnp.exp(sim)

        sel1f = jnp.where(imp_all[:, 2 * h + 1:2 * h + 2] >
                          imp_all[:, 2 * h:2 * h + 1], 1.0, 0.0).astype(bf16)
        msel16 = inv_colhalf16 + sel1f * diff16

        ew = e * band16
        es = e * msel16
        vext = jnp.concatenate([vb[:, sl], ones_col], axis=1)
        o_w = jnp.dot(ew, vext, preferred_element_type=f32)
        o_s = jnp.dot(es, vext, preferred_element_type=f32)
        gw_col = gates[:, 3 * h + 2:3 * h + 3] / o_w[:, DH:DH + 1]
        gs_col = gates[:, 3 * h + 1:3 * h + 2] / o_s[:, DH:DH + 1]
        out_h = gw_col * o_w[:, :DH] + gs_col * o_s[:, :DH]

        kc = jnp.dot(poolk16[:, sl], wkc_ref[...],
                     preferred_element_type=f32).astype(bf16)
        vc = jnp.dot(poolv16[:, sl], wvc_ref[...],
                     preferred_element_type=f32).astype(bf16)
        sim_c = jax.lax.dot_general(qsb[:, sl], kc, DN,
                                    preferred_element_type=f32)
        p_c = jnp.exp(sim_c)
        p_c = (p_c / jnp.sum(p_c, axis=-1, keepdims=True)).astype(bf16)
        out_c = jnp.dot(p_c, vc, preferred_element_type=f32)

        outs.append(gates[:, 3 * h:3 * h + 1] * out_c + out_h)

    attn = jnp.concatenate(outs, axis=-1).astype(bf16)

    y = jnp.dot(attn, wo_ref[...], preferred_element_type=f32) + x
    mu = jnp.mean(y, axis=-1, keepdims=True)
    var = jnp.mean(y * y, axis=-1, keepdims=True) - mu * mu
    ln = (y - mu) * jax.lax.rsqrt(var + 1e-5)
    hmid = jax.lax.dot_general(ln.astype(bf16), w1t_ref[...], DN,
                               preferred_element_type=f32)
    hmid = jax.nn.gelu((hmid + b1_ref[...]).astype(bf16))
    res = jax.lax.dot_general(hmid, w2t_ref[...], DN,
                              preferred_element_type=f32) + b2_ref[...] + y
    if last:
        out_ref[0] = jnp.transpose(res)
    else:
        out_ref[0] = res


@functools.partial(jax.jit, static_argnames=())
def kernel(x, pos_emb, g, Wq, Wk, Wv, Wkc, Wvc, Wg, Wo, ln_g, ln_b,
           W1, b1, W2, b2):
    b, c, h, w = x.shape
    tok = x.reshape(b, c, N)

    whole = lambda *dims: pl.BlockSpec(dims, lambda bi: (0,) * len(dims))
    for i in range(DEPTH):
        first = i == 0
        last = i == DEPTH - 1
        tok_spec = (pl.BlockSpec((1, DIM, N), lambda bi: (bi, 0, 0)) if first
                    else pl.BlockSpec((1, N, DIM), lambda bi: (bi, 0, 0)))
        out_spec = (pl.BlockSpec((1, DIM, N), lambda bi: (bi, 0, 0)) if last
                    else pl.BlockSpec((1, N, DIM), lambda bi: (bi, 0, 0)))
        out_shape = (jax.ShapeDtypeStruct((B, DIM, N), f32) if last
                     else jax.ShapeDtypeStruct((B, N, DIM), f32))
        pos_in = ([pos_emb[:N].reshape(1, N)], [whole(1, N)]) if first \
            else ([], [])
        tok = pl.pallas_call(
            functools.partial(_layer_body, first, last),
            grid=(B,),
            in_specs=[tok_spec] + pos_in[1] + [
                whole(DIM, DIM), whole(DIM, DIM), whole(DIM, DIM),
                whole(DIM, HEADS * 3),
                whole(DH, DH), whole(DH, DH),
                whole(DIM, DIM),
                whole(MLP, DIM), whole(1, MLP),
                whole(DIM, MLP), whole(1, DIM),
            ],
            out_specs=out_spec,
            out_shape=out_shape,
            compiler_params=pltpu.CompilerParams(
                dimension_semantics=("parallel",)),
        )(tok, *pos_in[0],
          (Wq[i] * g[i][:, None]).astype(bf16),
          (Wk[i] * g[i][:, None]).astype(bf16),
          (Wv[i] * g[i][:, None]).astype(bf16),
          (Wg[i] * g[i][:, None]).astype(bf16),
          Wkc[i].astype(bf16), Wvc[i].astype(bf16),
          Wo[i].astype(bf16),
          (W1[i] * ln_g[i][None, :]).astype(bf16),
          (b1[i] + W1[i] @ ln_b[i]).reshape(1, MLP),
          W2[i].astype(bf16), b2[i].reshape(1, DIM))

    return tok.reshape(b, c, h, w)
